# Initial kernel scaffold; baseline (speedup 1.0000x reference)
import jax
import jax.numpy as jnp
from jax.experimental import pallas as pl


def kernel(seq, emb_weight):
    raise NotImplementedError("write your pallas kernel here")


# speedup vs baseline: 1.6269x; 1.6269x over previous
import functools

import jax
import jax.numpy as jnp
from jax import lax
from jax.experimental import pallas as pl
from jax.experimental.pallas import tpu as pltpu
from jax.experimental.pallas import tpu_sc as plsc


@functools.lru_cache(maxsize=None)
def _build_lookup(N: int, V: int, D: int):
    info = plsc.get_sparse_core_info()
    NC, NS, L = info.num_cores, info.num_subcores, info.num_lanes
    NW = NC * NS
    assert D == L and N % NW == 0
    per_w = N // NW
    C = 2048
    while per_w % C != 0:
        C //= 2
    iters = per_w // C

    mesh = plsc.VectorSubcoreMesh(core_axis_name="c", subcore_axis_name="s")

    @functools.partial(
        pl.kernel,
        mesh=mesh,
        out_type=jax.ShapeDtypeStruct((N, D), jnp.float32),
        scratch_types=[
            pltpu.VMEM((C,), jnp.int32),
            pltpu.VMEM((C, D), jnp.float32),
            pltpu.SemaphoreType.DMA,
        ],
        compiler_params=pltpu.CompilerParams(use_tc_tiling_on_sc=False),
    )
    def lookup(seq_hbm, table_hbm, out_hbm, idx_v, rows_v, sem):
        wid = lax.axis_index("s") * NC + lax.axis_index("c")
        base = wid * per_w

        def step(g, carry):
            off = base + g * C
            pltpu.sync_copy(seq_hbm.at[pl.ds(off, C)], idx_v)
            pltpu.async_copy(table_hbm.at[idx_v], rows_v, sem).wait()
            pltpu.sync_copy(rows_v, out_hbm.at[pl.ds(off, C)])
            return carry

        lax.fori_loop(0, iters, step, 0)

    return lookup


def kernel(seq, emb_weight):
    B, T = seq.shape
    V, D = emb_weight.shape
    N = B * T
    flat = seq.reshape(N).astype(jnp.int32)
    out = _build_lookup(N, V, D)(flat, emb_weight)
    return out.reshape(B, T, D)


# speedup vs baseline: 5.9500x; 3.6573x over previous
import functools

import jax
import jax.numpy as jnp
from jax import lax
from jax.experimental import pallas as pl
from jax.experimental.pallas import tpu as pltpu
from jax.experimental.pallas import tpu_sc as plsc


@functools.lru_cache(maxsize=None)
def _build_lookup(N: int, V: int, D: int):
    info = plsc.get_sparse_core_info()
    NC, NS, L = info.num_cores, info.num_subcores, info.num_lanes
    NW = NC * NS
    assert D == L and N % NW == 0
    per_w = N // NW

    C = 3200
    while per_w % (2 * C) != 0:
        C //= 2
    iters = per_w // C
    pairs = iters // 2
    G = C // L

    mesh = plsc.VectorSubcoreMesh(core_axis_name="c", subcore_axis_name="s")

    @functools.partial(
        pl.kernel,
        mesh=mesh,
        out_type=jax.ShapeDtypeStruct((N * D,), jnp.float32),
        scratch_types=[
            pltpu.VMEM((V, D), jnp.float32),
            pltpu.VMEM((C,), jnp.int32),
            pltpu.VMEM((C,), jnp.int32),
            pltpu.VMEM((C * D,), jnp.float32),
            pltpu.VMEM((C * D,), jnp.float32),
            pltpu.SemaphoreType.DMA,
            pltpu.SemaphoreType.DMA,
            pltpu.SemaphoreType.DMA,
            pltpu.SemaphoreType.DMA,
        ],
        compiler_params=pltpu.CompilerParams(
            use_tc_tiling_on_sc=False, needs_layout_passes=False),
    )
    def lookup(seq_hbm, table_hbm, out_hbm, table_v, idx0, idx1, rows0, rows1,
               sem_i0, sem_i1, sem_o0, sem_o1):
        wid = lax.axis_index("s") * NC + lax.axis_index("c")
        base = wid * per_w
        iota = lax.iota(jnp.int32, L)

        pltpu.sync_copy(table_hbm, table_v)
        pltpu.async_copy(seq_hbm.at[pl.ds(base, C)], idx0, sem_i0)
        pltpu.async_copy(seq_hbm.at[pl.ds(base + C, C)], idx1, sem_i1)

        def expand(idx_ref, rows_ref):
            def group(j, carry):
                seqv = idx_ref[pl.ds(j * L, L)]
                for t in range(L):
                    s = jnp.take(seqv, jnp.full((L,), t, jnp.int32))
                    row = plsc.load_gather(table_v, [s, iota])
                    rows_ref[pl.ds((j * L + t) * L, L)] = row
                return carry

            lax.fori_loop(0, G, group, 0, unroll=4)

        def half(g, idx_ref, rows_ref, sem_i, sem_o):
            pltpu.make_async_copy(
                seq_hbm.at[pl.ds(base, C)], idx_ref, sem_i).wait()

            @pl.when(g >= 2)
            def _():
                pltpu.make_async_copy(
                    rows_ref, out_hbm.at[pl.ds(base * D, C * D)], sem_o).wait()

            expand(idx_ref, rows_ref)
            pltpu.async_copy(
                rows_ref, out_hbm.at[pl.ds((base + g * C) * D, C * D)], sem_o)

            @pl.when(g + 2 < iters)
            def _():
                pltpu.async_copy(
                    seq_hbm.at[pl.ds(base + (g + 2) * C, C)], idx_ref, sem_i)

        def pair(p, carry):
            half(2 * p, idx0, rows0, sem_i0, sem_o0)
            half(2 * p + 1, idx1, rows1, sem_i1, sem_o1)
            return carry

        lax.fori_loop(0, pairs, pair, 0)
        pltpu.make_async_copy(
            rows0, out_hbm.at[pl.ds(base * D, C * D)], sem_o0).wait()
        pltpu.make_async_copy(
            rows1, out_hbm.at[pl.ds(base * D, C * D)], sem_o1).wait()

    return lookup


def kernel(seq, emb_weight):
    B, T = seq.shape
    V, D = emb_weight.shape
    N = B * T
    flat = seq.reshape(N).astype(jnp.int32)
    out = _build_lookup(N, V, D)(flat, emb_weight)
    return out.reshape(B, T, D)
